# baseline (speedup 1.0000x reference)
import functools

import jax
import jax.numpy as jnp
from jax import lax
from jax.experimental import pallas as pl
from jax.experimental.pallas import tpu as pltpu
from jax.experimental.pallas import tpu_sc as plsc

F32 = jnp.float32
N = 4096
E = 131072
D_NODE = 128
D_EDGE = 16
H = 64
OUT = 64
EMB = 2


def _dot(a, b):
    return lax.dot_general(
        a, b, (((a.ndim - 1,), (0,)), ((), ())),
        precision=lax.Precision.HIGHEST, preferred_element_type=F32)


def _blockdiag(w, k):
    return jax.scipy.linalg.block_diag(*([w] * k))


NC = 2
NS = 16
NW = NC * NS
EPW = E // NW
CH = 64
NCHUNK = EPW // CH
NSTRIPE = N // NS


def _sc_edge_body(pq_hbm, r_hbm, ei_hbm,
                  relu_out, spart_out,
                  rowi, coli, rows_s, pqr, pqc, rbuf, hbuf64, hbuf,
                  sem_i, sem_p0, sem_p1, sem_q0, sem_q1, sem_r0, sem_r1,
                  sem_s0, sem_s1, sem_o0, sem_o1, s_sh):
    sems_p = (sem_p0, sem_p1)
    sems_q = (sem_q0, sem_q1)
    sems_r = (sem_r0, sem_r1)
    sems_s = (sem_s0, sem_s1)
    sems_o = (sem_o0, sem_o1)
    cid = lax.axis_index("c")
    sid = lax.axis_index("s")
    wid = sid * NC + cid

    def _zrow(i, _):
        for k in range(8):
            hbuf[0, i, pl.ds(k * 16, 16)] = jnp.zeros((16,), F32)
        return 0
    lax.fori_loop(0, CH, _zrow, 0)
    base_n = sid * NSTRIPE
    for z in range(NSTRIPE // CH):
        pltpu.sync_copy(hbuf.at[0], s_sh.at[pl.ds(base_n + z * CH, CH)])

    def _orow(i, _):
        hbuf[0, i, pl.ds(H, 16)] = jnp.ones((16,), F32)
        hbuf[1, i, pl.ds(H, 16)] = jnp.ones((16,), F32)
        for k in range(5, 8):
            hbuf[1, i, pl.ds(k * 16, 16)] = jnp.zeros((16,), F32)
        return 0
    lax.fori_loop(0, CH, _orow, 0)
    plsc.subcore_barrier()

    ebase = wid * EPW

    def _idx_copy(c, b):
        base = ebase + c * CH
        pltpu.async_copy(ei_hbm.at[0, pl.ds(base, CH)], rowi.at[b], sem_i)
        pltpu.async_copy(ei_hbm.at[1, pl.ds(base, CH)], coli.at[b], sem_i)

    def _gathers(c, b):
        base = ebase + c * CH
        pltpu.async_copy(pq_hbm.at[rowi.at[b]], pqr.at[b], sems_p[b])
        pltpu.async_copy(pq_hbm.at[coli.at[b]], pqc.at[b], sems_q[b])
        pltpu.async_copy(
            r_hbm.at[pl.ds(pl.multiple_of(base // 8, 8), CH // 8)],
            rbuf.at[b], sems_r[b])

    def _wait_gathers(b):
        pltpu.make_async_copy(
            pq_hbm.at[rowi.at[b]], pqr.at[b], sems_p[b]).wait()
        pltpu.make_async_copy(
            pq_hbm.at[coli.at[b]], pqc.at[b], sems_q[b]).wait()
        pltpu.make_async_copy(
            r_hbm.at[pl.ds(0, CH // 8)], rbuf.at[b], sems_r[b]).wait()

    def _wait_stores(b):
        pltpu.make_async_copy(
            hbuf.at[b], s_sh.at[rows_s.at[b]], sems_s[b]).wait()
        pltpu.make_async_copy(
            hbuf64.at[b], relu_out.at[pl.ds(0, CH // 8)], sems_o[b]).wait()

    _idx_copy(0, 0)
    _idx_copy(1, 1)
    pltpu.make_async_copy(ei_hbm.at[0, pl.ds(0, CH)], rowi.at[0], sem_i).wait()
    pltpu.make_async_copy(ei_hbm.at[0, pl.ds(0, CH)], coli.at[0], sem_i).wait()
    pltpu.make_async_copy(ei_hbm.at[0, pl.ds(0, CH)], rowi.at[1], sem_i).wait()
    pltpu.make_async_copy(ei_hbm.at[0, pl.ds(0, CH)], coli.at[1], sem_i).wait()
    _gathers(0, 0)
    _gathers(1, 1)

    def _pair(t, _):
        for b in range(2):
            c = 2 * t + b
            base = ebase + c * CH

            @pl.when(t > 0)
            def _():
                _wait_stores(b)

            _wait_gathers(b)

            for i2 in range(CH // 16):
                sl16 = pl.ds(i2 * 16, 16)
                rows_s[b, sl16] = rowi[b, sl16]

            @pl.when(t < NCHUNK // 2 - 1)
            def _():
                _idx_copy(c + 2, b)

            def _crow(i, _2):
                ip = i // 8
                io = (i % 8) * H
                for k in range(4):
                    sl = pl.ds(k * 16, 16)
                    slp = pl.ds(io + k * 16, 16)
                    v = jnp.maximum(
                        pqr[b, i, sl] + pqc[b, i, pl.ds(H + k * 16, 16)]
                        + rbuf[b, ip, slp], 0.0)
                    hbuf[b, i, sl] = v
                    hbuf64[b, ip, slp] = v
                return 0
            lax.fori_loop(0, CH, _crow, 0)

            pltpu.async_copy(hbuf.at[b], s_sh.at[rows_s.at[b]], sems_s[b],
                             add=True)
            pltpu.async_copy(
                hbuf64.at[b],
                relu_out.at[pl.ds(pl.multiple_of(base // 8, 8), CH // 8)],
                sems_o[b])

            @pl.when(t < NCHUNK // 2 - 1)
            def _():
                pltpu.make_async_copy(ei_hbm.at[0, pl.ds(0, CH)],
                                      rowi.at[b], sem_i).wait()
                pltpu.make_async_copy(ei_hbm.at[0, pl.ds(0, CH)],
                                      coli.at[b], sem_i).wait()
                _gathers(c + 2, b)
        return 0
    lax.fori_loop(0, NCHUNK // 2, _pair, 0)

    _wait_stores(0)
    _wait_stores(1)
    plsc.subcore_barrier()
    pltpu.sync_copy(s_sh.at[pl.ds(base_n, NSTRIPE)],
                    spart_out.at[cid, pl.ds(base_n, NSTRIPE)])


def _sc_edge(pq, r, edge_index):
    mesh = plsc.VectorSubcoreMesh(core_axis_name="c", subcore_axis_name="s")
    fn = functools.partial(
        pl.kernel,
        mesh=mesh,
        out_type=[jax.ShapeDtypeStruct((E // 8, 8 * H), F32),
                  jax.ShapeDtypeStruct((NC, N, 2 * H), F32)],
        scratch_types=[
            pltpu.VMEM((2, CH), jnp.int32),
            pltpu.VMEM((2, CH), jnp.int32),
            pltpu.VMEM((2, CH), jnp.int32),
            pltpu.VMEM((2, CH, 2 * H), F32),
            pltpu.VMEM((2, CH, 2 * H), F32),
            pltpu.VMEM((2, CH // 8, 8 * H), F32),
            pltpu.VMEM((2, CH // 8, 8 * H), F32),
            pltpu.VMEM((2, CH, 2 * H), F32),
        ] + [pltpu.SemaphoreType.DMA] * 11 + [
            pltpu.VMEM_SHARED((N, 2 * H), F32),
        ],
    )(_sc_edge_body)
    return fn(pq, r, edge_index)


def _pre_body(nf_ref, w1eab_ref, pq_ref):
    pq_ref[...] = _dot(nf_ref[...], w1eab_ref[...])


def _r_body(ea_ref, w_ref, b_ref, nf_ref, w1eab_ref, r_ref, pq_ref):
    r_ref[...] = _dot(ea_ref[...], w_ref[...]) + b_ref[...]

    @pl.when(pl.program_id(0) == 0)
    def _():
        pq_ref[...] = _dot(nf_ref[...], w1eab_ref[...])


def _epost_body(h_ref, wemb_ref, bemb_ref, wrec_ref, brec_ref,
                emb_ref, rec_ref):
    emb = _dot(h_ref[...], wemb_ref[...]) + bemb_ref[...]
    emb_ref[...] = emb
    rec_ref[...] = _dot(emb, wrec_ref[...]) + brec_ref[...]


def _node_body(sp_ref, nf_ref, w2e_ref, b2e_ref, w1n_ref, b1n_ref,
               w2n_ref, b2n_ref, wne_ref, bne_ref, wdn_ref, bdn_ref,
               emb_ref, rec_ref):
    sp = sp_ref[...]
    sboth = sp[0] + sp[1]
    s = sboth[:, :H]
    cnt = sboth[:, H:H + 1]
    agg = _dot(s, w2e_ref[...]) + cnt * b2e_ref[...]
    node_in = jnp.concatenate([nf_ref[...], agg], axis=1)
    hidden = jnp.maximum(_dot(node_in, w1n_ref[...]) + b1n_ref[...], 0.0)
    node_out = _dot(hidden, w2n_ref[...]) + b2n_ref[...]
    emb = _dot(node_out, wne_ref[...]) + bne_ref[...]
    emb_ref[...] = emb
    rec_ref[...] = _dot(emb, wdn_ref[...]) + bdn_ref[...]


def _adj_body(emb_ref, embt_ref, o_ref, *, bm):
    i = pl.program_id(0)
    a = emb_ref[...]
    bt = embt_ref[...]
    d0 = a[:, 0:1] - bt[0:1, :]
    d1 = a[:, 1:2] - bt[1:2, :]
    s = jax.nn.sigmoid(3.0 * (d0 * d0 + d1 * d1) - 1.0)
    rows = lax.broadcasted_iota(jnp.int32, (bm, N), 0) + i * bm
    cols = lax.broadcasted_iota(jnp.int32, (bm, N), 1)
    o_ref[...] = jnp.where(rows == cols, 0.0, s)


def kernel(node_feats, edge_index, edge_attr, W1e, b1e, W2e, b2e,
           W1n, b1n, W2n, b2n, Wne, bne, Wee, bee, Wdn, bdn, Wde, bde):
    w1ea = W1e[:D_NODE]
    w1eb = W1e[D_NODE:2 * D_NODE]
    w1ec = W1e[2 * D_NODE:]

    ea_packed = edge_attr.reshape(E // 8, 8 * D_EDGE)
    BR = 4096
    r, pq = pl.pallas_call(
        _r_body,
        grid=(E // 8 // BR,),
        in_specs=[pl.BlockSpec((BR, 8 * D_EDGE), lambda i: (i, 0)),
                  pl.BlockSpec((8 * D_EDGE, 8 * H), lambda i: (0, 0)),
                  pl.BlockSpec((1, 8 * H), lambda i: (0, 0)),
                  pl.BlockSpec((N, D_NODE), lambda i: (0, 0)),
                  pl.BlockSpec((D_NODE, 2 * H), lambda i: (0, 0))],
        out_specs=[pl.BlockSpec((BR, 8 * H), lambda i: (i, 0)),
                   pl.BlockSpec((N, 2 * H), lambda i: (0, 0))],
        out_shape=[jax.ShapeDtypeStruct((E // 8, 8 * H), F32),
                   jax.ShapeDtypeStruct((N, 2 * H), F32)],
    )(ea_packed, _blockdiag(w1ec, 8), jnp.tile(b1e, 8).reshape(1, 8 * H),
      node_feats, jnp.concatenate([w1ea, w1eb], axis=1))

    relu_h, sp = _sc_edge(pq, r, edge_index)

    we2 = lax.dot_general(W2e, Wee, (((1,), (0,)), ((), ())),
                          precision=lax.Precision.HIGHEST,
                          preferred_element_type=F32)
    be2 = b2e @ Wee + bee

    BP = 2048
    emb_pk, rec_pk = pl.pallas_call(
        _epost_body,
        grid=(E // 8 // BP,),
        in_specs=[pl.BlockSpec((BP, 8 * H), lambda i: (i, 0)),
                  pl.BlockSpec((8 * H, 8 * EMB), lambda i: (0, 0)),
                  pl.BlockSpec((1, 8 * EMB), lambda i: (0, 0)),
                  pl.BlockSpec((8 * EMB, 8 * D_EDGE), lambda i: (0, 0)),
                  pl.BlockSpec((1, 8 * D_EDGE), lambda i: (0, 0))],
        out_specs=[pl.BlockSpec((BP, 8 * EMB), lambda i: (i, 0)),
                   pl.BlockSpec((BP, 8 * D_EDGE), lambda i: (i, 0))],
        out_shape=[jax.ShapeDtypeStruct((E // 8, 8 * EMB), F32),
                   jax.ShapeDtypeStruct((E // 8, 8 * D_EDGE), F32)],
    )(relu_h, _blockdiag(we2, 8),
      jnp.tile(be2, 8).reshape(1, 8 * EMB), _blockdiag(Wde, 8),
      jnp.tile(bde, 8).reshape(1, 8 * D_EDGE))
    edge_emb = emb_pk.reshape(E, EMB)
    recon_edge = rec_pk.reshape(E, D_EDGE)

    node_emb, recon_node = pl.pallas_call(
        _node_body,
        out_shape=[jax.ShapeDtypeStruct((N, EMB), F32),
                   jax.ShapeDtypeStruct((N, D_NODE), F32)],
    )(sp, node_feats, W2e, b2e.reshape(1, OUT), W1n, b1n.reshape(1, H),
      W2n, b2n.reshape(1, OUT), Wne, bne.reshape(1, EMB),
      Wdn, bdn.reshape(1, D_NODE))

    BM = 512
    adj = pl.pallas_call(
        functools.partial(_adj_body, bm=BM),
        grid=(N // BM,),
        in_specs=[pl.BlockSpec((BM, EMB), lambda i: (i, 0)),
                  pl.BlockSpec((EMB, N), lambda i: (0, 0))],
        out_specs=pl.BlockSpec((BM, N), lambda i: (i, 0)),
        out_shape=jax.ShapeDtypeStruct((N, N), F32),
    )(node_emb, node_emb.T)

    return (node_emb, edge_emb, recon_node, recon_edge, adj)


# speedup vs baseline: 1.0053x; 1.0053x over previous
import functools

import jax
import jax.numpy as jnp
from jax import lax
from jax.experimental import pallas as pl
from jax.experimental.pallas import tpu as pltpu
from jax.experimental.pallas import tpu_sc as plsc

F32 = jnp.float32
N = 4096
E = 131072
D_NODE = 128
D_EDGE = 16
H = 64
OUT = 64
EMB = 2


def _dot(a, b):
    return lax.dot_general(
        a, b, (((a.ndim - 1,), (0,)), ((), ())),
        precision=lax.Precision.HIGHEST, preferred_element_type=F32)


def _blockdiag(w, k):
    return jax.scipy.linalg.block_diag(*([w] * k))


NC = 2
NS = 16
NW = NC * NS
EPW = E // NW
CH = 64
NCHUNK = EPW // CH
NSTRIPE = N // NS


def _sc_edge_body(pq_hbm, r_hbm, ei_hbm,
                  relu_out, spart_out,
                  rowi, coli, rows_s, pqr, pqc, rbuf, hbuf64, hbuf,
                  sem_i, sem_p0, sem_p1, sem_q0, sem_q1, sem_r0, sem_r1,
                  sem_s0, sem_s1, sem_o0, sem_o1, s_sh):
    sems_p = (sem_p0, sem_p1)
    sems_q = (sem_q0, sem_q1)
    sems_r = (sem_r0, sem_r1)
    sems_s = (sem_s0, sem_s1)
    sems_o = (sem_o0, sem_o1)
    cid = lax.axis_index("c")
    sid = lax.axis_index("s")
    wid = sid * NC + cid

    def _zrow(i, _):
        for k in range(8):
            hbuf[0, i, pl.ds(k * 16, 16)] = jnp.zeros((16,), F32)
        return 0
    lax.fori_loop(0, CH, _zrow, 0)
    base_n = sid * NSTRIPE
    for z in range(NSTRIPE // CH):
        pltpu.sync_copy(hbuf.at[0], s_sh.at[pl.ds(base_n + z * CH, CH)])

    def _orow(i, _):
        hbuf[0, i, pl.ds(H, 16)] = jnp.ones((16,), F32)
        hbuf[1, i, pl.ds(H, 16)] = jnp.ones((16,), F32)
        for k in range(5, 8):
            hbuf[1, i, pl.ds(k * 16, 16)] = jnp.zeros((16,), F32)
        return 0
    lax.fori_loop(0, CH, _orow, 0)
    plsc.subcore_barrier()

    ebase = wid * EPW

    def _idx_copy(c, b):
        base = ebase + c * CH
        pltpu.async_copy(ei_hbm.at[0, pl.ds(base, CH)], rowi.at[b], sem_i)
        pltpu.async_copy(ei_hbm.at[1, pl.ds(base, CH)], coli.at[b], sem_i)

    def _gathers(c, b):
        base = ebase + c * CH
        pltpu.async_copy(pq_hbm.at[rowi.at[b]], pqr.at[b], sems_p[b])
        pltpu.async_copy(pq_hbm.at[coli.at[b]], pqc.at[b], sems_q[b])
        pltpu.async_copy(
            r_hbm.at[pl.ds(pl.multiple_of(base // 8, 8), CH // 8)],
            rbuf.at[b], sems_r[b])

    def _wait_gathers(b):
        pltpu.make_async_copy(
            pq_hbm.at[rowi.at[b]], pqr.at[b], sems_p[b]).wait()
        pltpu.make_async_copy(
            pq_hbm.at[coli.at[b]], pqc.at[b], sems_q[b]).wait()
        pltpu.make_async_copy(
            r_hbm.at[pl.ds(0, CH // 8)], rbuf.at[b], sems_r[b]).wait()

    def _wait_stores(b):
        pltpu.make_async_copy(
            hbuf.at[b], s_sh.at[rows_s.at[b]], sems_s[b]).wait()
        pltpu.make_async_copy(
            hbuf64.at[b], relu_out.at[pl.ds(0, CH // 8)], sems_o[b]).wait()

    _idx_copy(0, 0)
    _idx_copy(1, 1)
    pltpu.make_async_copy(ei_hbm.at[0, pl.ds(0, CH)], rowi.at[0], sem_i).wait()
    pltpu.make_async_copy(ei_hbm.at[0, pl.ds(0, CH)], coli.at[0], sem_i).wait()
    pltpu.make_async_copy(ei_hbm.at[0, pl.ds(0, CH)], rowi.at[1], sem_i).wait()
    pltpu.make_async_copy(ei_hbm.at[0, pl.ds(0, CH)], coli.at[1], sem_i).wait()
    _gathers(0, 0)
    _gathers(1, 1)

    def _pair(t, _):
        for b in range(2):
            c = 2 * t + b
            base = ebase + c * CH

            @pl.when(t > 0)
            def _():
                _wait_stores(b)

            _wait_gathers(b)

            for i2 in range(CH // 16):
                sl16 = pl.ds(i2 * 16, 16)
                rows_s[b, sl16] = rowi[b, sl16]

            @pl.when(t < NCHUNK // 2 - 1)
            def _():
                _idx_copy(c + 2, b)

            def _crow(ip, _2):
                for io in range(8):
                    i = ip * 8 + io
                    for k in range(4):
                        sl = pl.ds(k * 16, 16)
                        slp = pl.ds(io * H + k * 16, 16)
                        v = jnp.maximum(
                            pqr[b, i, sl] + pqc[b, i, pl.ds(H + k * 16, 16)]
                            + rbuf[b, ip, slp], 0.0)
                        hbuf[b, i, sl] = v
                        hbuf64[b, ip, slp] = v
                return 0
            lax.fori_loop(0, CH // 8, _crow, 0)

            pltpu.async_copy(hbuf.at[b], s_sh.at[rows_s.at[b]], sems_s[b],
                             add=True)
            pltpu.async_copy(
                hbuf64.at[b],
                relu_out.at[pl.ds(pl.multiple_of(base // 8, 8), CH // 8)],
                sems_o[b])

            @pl.when(t < NCHUNK // 2 - 1)
            def _():
                pltpu.make_async_copy(ei_hbm.at[0, pl.ds(0, CH)],
                                      rowi.at[b], sem_i).wait()
                pltpu.make_async_copy(ei_hbm.at[0, pl.ds(0, CH)],
                                      coli.at[b], sem_i).wait()
                _gathers(c + 2, b)
        return 0
    lax.fori_loop(0, NCHUNK // 2, _pair, 0)

    _wait_stores(0)
    _wait_stores(1)
    plsc.subcore_barrier()
    pltpu.sync_copy(s_sh.at[pl.ds(base_n, NSTRIPE)],
                    spart_out.at[cid, pl.ds(base_n, NSTRIPE)])


def _sc_edge(pq, r, edge_index):
    mesh = plsc.VectorSubcoreMesh(core_axis_name="c", subcore_axis_name="s")
    fn = functools.partial(
        pl.kernel,
        mesh=mesh,
        out_type=[jax.ShapeDtypeStruct((E // 8, 8 * H), F32),
                  jax.ShapeDtypeStruct((NC, N, 2 * H), F32)],
        scratch_types=[
            pltpu.VMEM((2, CH), jnp.int32),
            pltpu.VMEM((2, CH), jnp.int32),
            pltpu.VMEM((2, CH), jnp.int32),
            pltpu.VMEM((2, CH, 2 * H), F32),
            pltpu.VMEM((2, CH, 2 * H), F32),
            pltpu.VMEM((2, CH // 8, 8 * H), F32),
            pltpu.VMEM((2, CH // 8, 8 * H), F32),
            pltpu.VMEM((2, CH, 2 * H), F32),
        ] + [pltpu.SemaphoreType.DMA] * 11 + [
            pltpu.VMEM_SHARED((N, 2 * H), F32),
        ],
    )(_sc_edge_body)
    return fn(pq, r, edge_index)


def _pre_body(nf_ref, w1eab_ref, pq_ref):
    pq_ref[...] = _dot(nf_ref[...], w1eab_ref[...])


def _r_body(ea_ref, w_ref, b_ref, r_ref):
    r_ref[...] = _dot(ea_ref[...], w_ref[...]) + b_ref[...]


def _epost_body(h_ref, wemb_ref, bemb_ref, wrec_ref, brec_ref,
                emb_ref, rec_ref):
    emb = _dot(h_ref[...], wemb_ref[...]) + bemb_ref[...]
    emb_ref[...] = emb
    rec_ref[...] = _dot(emb, wrec_ref[...]) + brec_ref[...]


def _node_body(sp_ref, nf_ref, w2e_ref, b2e_ref, w1n_ref, b1n_ref,
               w2n_ref, b2n_ref, wne_ref, bne_ref, wdn_ref, bdn_ref,
               emb_ref, rec_ref):
    sp = sp_ref[...]
    sboth = sp[0] + sp[1]
    s = sboth[:, :H]
    cnt = sboth[:, H:H + 1]
    agg = _dot(s, w2e_ref[...]) + cnt * b2e_ref[...]
    node_in = jnp.concatenate([nf_ref[...], agg], axis=1)
    hidden = jnp.maximum(_dot(node_in, w1n_ref[...]) + b1n_ref[...], 0.0)
    node_out = _dot(hidden, w2n_ref[...]) + b2n_ref[...]
    emb = _dot(node_out, wne_ref[...]) + bne_ref[...]
    emb_ref[...] = emb
    rec_ref[...] = _dot(emb, wdn_ref[...]) + bdn_ref[...]


def _adj_body(emb_ref, embt_ref, o_ref, *, bm):
    i = pl.program_id(0)
    a = emb_ref[...]
    bt = embt_ref[...]
    d0 = a[:, 0:1] - bt[0:1, :]
    d1 = a[:, 1:2] - bt[1:2, :]
    s = jax.nn.sigmoid(3.0 * (d0 * d0 + d1 * d1) - 1.0)
    rows = lax.broadcasted_iota(jnp.int32, (bm, N), 0) + i * bm
    cols = lax.broadcasted_iota(jnp.int32, (bm, N), 1)
    o_ref[...] = jnp.where(rows == cols, 0.0, s)


def kernel(node_feats, edge_index, edge_attr, W1e, b1e, W2e, b2e,
           W1n, b1n, W2n, b2n, Wne, bne, Wee, bee, Wdn, bdn, Wde, bde):
    w1ea = W1e[:D_NODE]
    w1eb = W1e[D_NODE:2 * D_NODE]
    w1ec = W1e[2 * D_NODE:]

    pq = pl.pallas_call(
        _pre_body,
        out_shape=jax.ShapeDtypeStruct((N, 2 * H), F32),
    )(node_feats, jnp.concatenate([w1ea, w1eb], axis=1))

    ea_packed = edge_attr.reshape(E // 8, 8 * D_EDGE)
    BR = 4096
    r = pl.pallas_call(
        _r_body,
        grid=(E // 8 // BR,),
        in_specs=[pl.BlockSpec((BR, 8 * D_EDGE), lambda i: (i, 0)),
                  pl.BlockSpec((8 * D_EDGE, 8 * H), lambda i: (0, 0)),
                  pl.BlockSpec((1, 8 * H), lambda i: (0, 0))],
        out_specs=pl.BlockSpec((BR, 8 * H), lambda i: (i, 0)),
        out_shape=jax.ShapeDtypeStruct((E // 8, 8 * H), F32),
    )(ea_packed, _blockdiag(w1ec, 8), jnp.tile(b1e, 8).reshape(1, 8 * H))

    relu_h, sp = _sc_edge(pq, r, edge_index)

    we2 = lax.dot_general(W2e, Wee, (((1,), (0,)), ((), ())),
                          precision=lax.Precision.HIGHEST,
                          preferred_element_type=F32)
    be2 = b2e @ Wee + bee

    BP = 2048
    emb_pk, rec_pk = pl.pallas_call(
        _epost_body,
        grid=(E // 8 // BP,),
        in_specs=[pl.BlockSpec((BP, 8 * H), lambda i: (i, 0)),
                  pl.BlockSpec((8 * H, 8 * EMB), lambda i: (0, 0)),
                  pl.BlockSpec((1, 8 * EMB), lambda i: (0, 0)),
                  pl.BlockSpec((8 * EMB, 8 * D_EDGE), lambda i: (0, 0)),
                  pl.BlockSpec((1, 8 * D_EDGE), lambda i: (0, 0))],
        out_specs=[pl.BlockSpec((BP, 8 * EMB), lambda i: (i, 0)),
                   pl.BlockSpec((BP, 8 * D_EDGE), lambda i: (i, 0))],
        out_shape=[jax.ShapeDtypeStruct((E // 8, 8 * EMB), F32),
                   jax.ShapeDtypeStruct((E // 8, 8 * D_EDGE), F32)],
    )(relu_h, _blockdiag(we2, 8),
      jnp.tile(be2, 8).reshape(1, 8 * EMB), _blockdiag(Wde, 8),
      jnp.tile(bde, 8).reshape(1, 8 * D_EDGE))
    edge_emb = emb_pk.reshape(E, EMB)
    recon_edge = rec_pk.reshape(E, D_EDGE)

    node_emb, recon_node = pl.pallas_call(
        _node_body,
        out_shape=[jax.ShapeDtypeStruct((N, EMB), F32),
                   jax.ShapeDtypeStruct((N, D_NODE), F32)],
    )(sp, node_feats, W2e, b2e.reshape(1, OUT), W1n, b1n.reshape(1, H),
      W2n, b2n.reshape(1, OUT), Wne, bne.reshape(1, EMB),
      Wdn, bdn.reshape(1, D_NODE))

    BM = 512
    adj = pl.pallas_call(
        functools.partial(_adj_body, bm=BM),
        grid=(N // BM,),
        in_specs=[pl.BlockSpec((BM, EMB), lambda i: (i, 0)),
                  pl.BlockSpec((EMB, N), lambda i: (0, 0))],
        out_specs=pl.BlockSpec((BM, N), lambda i: (i, 0)),
        out_shape=jax.ShapeDtypeStruct((N, N), F32),
    )(node_emb, node_emb.T)

    return (node_emb, edge_emb, recon_node, recon_edge, adj)
